# baseline (speedup 1.0000x reference)
import jax
import jax.numpy as jnp
from jax import lax
from jax.experimental import pallas as pl
from jax.experimental.pallas import tpu as pltpu
from jax.experimental.pallas import tpu_sc as plsc

_N = 10000
_NPAD = 10240
_E = 320000
_NC = 2
_NS = 16
_NW = _NC * _NS
_K = 80
_ZROW = 10000
_BN_EPS = 1e-5

_sc_mesh = plsc.VectorSubcoreMesh(
    core_axis_name="c", subcore_axis_name="s", num_cores=_NC, num_subcores=_NS
)

_EPT = _E // _NS
_NCH = _EPT // _K


_DROWS = 2560
_DACC = _DROWS + 128
_DRPS = _DACC // _NS


def _deg_body(dst_h, out_h, dstv, rows, zbuf, acc):
  c = lax.axis_index("c")
  s = lax.axis_index("s")

  zero = jnp.zeros((16,), jnp.float32)
  one = jnp.ones((16,), jnp.float32)
  def zfill(r, _):
    for cc in range(8):
      zbuf[r, pl.ds(cc * 16, 16)] = zero
    return 0
  lax.fori_loop(0, _DRPS, zfill, 0)
  def ofill(r, _):
    for cc in range(8):
      rows[r, pl.ds(cc * 16, 16)] = one
    return 0
  lax.fori_loop(0, _K, ofill, 0)

  base = s * _DRPS
  trash = jnp.full((16,), _DROWS, jnp.int32)
  nloc = jnp.full((16,), _DROWS, jnp.int32)

  for r in range(2):
    pltpu.sync_copy(dst_h.at[s], dstv)
    off = jnp.full((16,), (2 * c + r) * _DROWS, jnp.int32)
    def remap(j, _):
      for cc in range(_K // 16):
        sl = pl.ds(cc * 16, 16)
        d = dstv[j, sl] - off
        m = (d >= 0) & (d < nloc)
        dstv[j, sl] = jnp.where(m, d, trash)
      return 0
    lax.fori_loop(0, _NCH, remap, 0)

    pltpu.sync_copy(zbuf, acc.at[pl.ds(base, _DRPS)])
    plsc.subcore_barrier()

    def chunk(j, _):
      pltpu.sync_copy(rows, acc.at[dstv.at[j]], add=True)
      return 0
    lax.fori_loop(0, _NCH, chunk, 0)
    plsc.subcore_barrier()

    pltpu.sync_copy(acc.at[pl.ds(base, _DRPS)],
                    out_h.at[c, r, pl.ds(base, _DRPS)])


_deg_pass = pl.kernel(
    _deg_body,
    out_type=jax.ShapeDtypeStruct((_NC, 2, _DACC, 128), jnp.float32),
    mesh=_sc_mesh,
    scratch_types=[
        pltpu.VMEM((_NCH, _K), jnp.int32),
        pltpu.VMEM((_K, 128), jnp.float32),
        pltpu.VMEM((_DRPS, 128), jnp.float32),
        pltpu.VMEM_SHARED((_DACC, 128), jnp.float32),
    ],
    name="sc_deg",
)


def _make_agg(rows_per_core, n_rounds):
  rps = rows_per_core // _NS

  def body(table, src_h, dst_h, out_h, srcv, dstv, rows, zbuf, acc, sem):
    c = lax.axis_index("c")
    s = lax.axis_index("s")

    zero = jnp.zeros((16,), jnp.float32)
    def zfill(r, _):
      for cc in range(8):
        zbuf[r, pl.ds(cc * 16, 16)] = zero
      return 0
    lax.fori_loop(0, rps, zfill, 0)

    base = s * rps
    zrow = jnp.full((16,), _ZROW, jnp.int32)
    lzero = jnp.zeros((16,), jnp.int32)
    nloc = jnp.full((16,), rows_per_core, jnp.int32)

    for r in range(n_rounds):
      pltpu.sync_copy(src_h.at[s], srcv)
      pltpu.sync_copy(dst_h.at[s], dstv)
      qoff = (n_rounds * c + r) * rows_per_core
      off = jnp.full((16,), qoff, jnp.int32)
      def remap(j, _):
        for cc in range(_K // 16):
          sl = pl.ds(cc * 16, 16)
          d = dstv[j, sl] - off
          m = (d >= 0) & (d < nloc)
          dstv[j, sl] = jnp.where(m, d, lzero)
          srcv[j, sl] = jnp.where(m, srcv[j, sl], zrow)
        return 0
      lax.fori_loop(0, _NCH, remap, 0)

      pltpu.sync_copy(zbuf, acc.at[pl.ds(base, rps)])
      plsc.subcore_barrier()

      def chunk(j, _):
        pltpu.async_copy(table.at[srcv.at[j]], rows, sem).wait()
        pltpu.sync_copy(rows, acc.at[dstv.at[j]], add=True)
        return 0
      lax.fori_loop(0, _NCH, chunk, 0)
      plsc.subcore_barrier()

      if n_rounds == 1:
        pltpu.sync_copy(acc.at[pl.ds(base, rps)], out_h.at[c, pl.ds(base, rps)])
      else:
        pltpu.sync_copy(acc.at[pl.ds(base, rps)],
                        out_h.at[c, r, pl.ds(base, rps)])

  if n_rounds == 1:
    oshape = (_NC, rows_per_core, 128)
  else:
    oshape = (_NC, n_rounds, rows_per_core, 128)
  return pl.kernel(
      body,
      out_type=jax.ShapeDtypeStruct(oshape, jnp.float32),
      mesh=_sc_mesh,
      scratch_types=[
          pltpu.VMEM((_NCH, _K), jnp.int32),
          pltpu.VMEM((_NCH, _K), jnp.int32),
          pltpu.VMEM((_K, 128), jnp.float32),
          pltpu.VMEM((rps, 128), jnp.float32),
          pltpu.VMEM_SHARED((rows_per_core, 128), jnp.float32),
          pltpu.SemaphoreType.DMA,
      ],
      name=f"sc_agg_{rows_per_core}x{n_rounds}",
  )


_QROWS = _NPAD // 4
_agg_pass = _make_agg(_QROWS, 2)


_RB1 = 1024
_GRID1 = _NPAD // _RB1
_RB2 = 512
_GRID2 = _NPAD // _RB2


_qsplit128 = pl.BlockSpec((1, 1, _RB2, 128),
                          lambda i: (i // 10, (i // 5) % 2, i % 5, 0))


def _prescale_body(degp, x, xs, dis):
  deg = degp[0, 0, :, 0:1] + 1.0
  d = lax.rsqrt(deg)
  dis[...] = d
  xs[...] = x[...] * d


def _prescale(degp, x):
  return pl.pallas_call(
      _prescale_body,
      grid=(_GRID2,),
      in_specs=[
          _qsplit128,
          pl.BlockSpec((_RB2, 128), lambda i: (i, 0)),
      ],
      out_specs=[
          pl.BlockSpec((_RB2, 128), lambda i: (i, 0)),
          pl.BlockSpec((_RB2, 1), lambda i: (i, 0)),
      ],
      out_shape=[
          jax.ShapeDtypeStruct((_NPAD, 128), jnp.float32),
          jax.ShapeDtypeStruct((_NPAD, 1), jnp.float32),
      ],
  )(degp, x)


def _mm_body(p1, xs, dis, w1s, c1, w2, h2s):
  agg1 = (p1[0, 0] + xs[...]) * dis[...]
  y1 = jnp.dot(agg1, w1s[...], preferred_element_type=jnp.float32) + c1[...]
  y1 = jnp.maximum(y1, 0.0)
  v = jnp.dot(y1, w2[...], preferred_element_type=jnp.float32) * dis[...]
  rid = lax.broadcasted_iota(jnp.int32, (_RB2, 1), 0) + pl.program_id(0) * _RB2
  h2s[...] = jnp.where(rid < _N, v, 0.0)


def _mm(p1, xs, dis, w1s, c1, w2):
  return pl.pallas_call(
      _mm_body,
      grid=(_GRID2,),
      in_specs=[
          _qsplit128,
          pl.BlockSpec((_RB2, 128), lambda i: (i, 0)),
          pl.BlockSpec((_RB2, 1), lambda i: (i, 0)),
          pl.BlockSpec((128, 256), lambda i: (0, 0)),
          pl.BlockSpec((1, 256), lambda i: (0, 0)),
          pl.BlockSpec((256, 128), lambda i: (0, 0)),
      ],
      out_specs=pl.BlockSpec((_RB2, 128), lambda i: (i, 0)),
      out_shape=jax.ShapeDtypeStruct((_NPAD, 128), jnp.float32),
  )(p1, xs, dis, w1s, c1, w2)


def _final_body(p2, h2s, dis, sv, cv, out):
  y2 = (p2[0, 0] + h2s[...]) * dis[...] * sv[...] + cv[...]
  m = jnp.max(y2, axis=1, keepdims=True)
  e = jnp.exp(y2 - m)
  lse = jnp.log(jnp.sum(e, axis=1, keepdims=True))
  out[...] = y2 - m - lse


def _final(p2, h2s, dis, sv, cv):
  return pl.pallas_call(
      _final_body,
      grid=(_GRID2,),
      in_specs=[
          pl.BlockSpec((1, 1, _RB2, 128),
                       lambda i: (i // 10, (i // 5) % 2, i % 5, 0)),
          pl.BlockSpec((_RB2, 128), lambda i: (i, 0)),
          pl.BlockSpec((_RB2, 1), lambda i: (i, 0)),
          pl.BlockSpec((1, 128), lambda i: (0, 0)),
          pl.BlockSpec((1, 128), lambda i: (0, 0)),
      ],
      out_specs=pl.BlockSpec((_RB2, 128), lambda i: (i, 0)),
      out_shape=jax.ShapeDtypeStruct((_NPAD, 128), jnp.float32),
  )(p2, h2s, dis, sv, cv)


def kernel(x, edge_index, W1, b1, W2, b2,
           bn1_gamma, bn1_beta, bn1_mean, bn1_var,
           bn2_gamma, bn2_beta, bn2_mean, bn2_var):
  src_w = edge_index[0].astype(jnp.int32).reshape(_NS, _NCH, _K)
  dst_w = edge_index[1].astype(jnp.int32).reshape(_NS, _NCH, _K)
  xp = jnp.pad(x, ((0, _NPAD - _N), (0, 0)))

  s1 = bn1_gamma * lax.rsqrt(bn1_var + _BN_EPS)
  c1 = (b1 * s1 + bn1_beta - bn1_mean * s1)[None, :]
  w1s = W1 * s1[None, :]
  s2 = bn2_gamma * lax.rsqrt(bn2_var + _BN_EPS)
  cv = (b2 * s2 + bn2_beta - bn2_mean * s2)[None, :]
  sv = s2[None, :]

  degp = _deg_pass(dst_w)
  xs, dis = _prescale(degp, xp)
  p1 = _agg_pass(xs, src_w, dst_w)
  h2s = _mm(p1, xs, dis, w1s, c1, W2)
  p2 = _agg_pass(h2s, src_w, dst_w)
  out = _final(p2, h2s, dis, sv, cv)
  return out[:_N]
